# Initial kernel scaffold; baseline (speedup 1.0000x reference)
import jax
import jax.numpy as jnp
from jax.experimental import pallas as pl


def kernel(x_author, x_paper, edge_index_ap, edge_index_pa, W_author, b_author, W_paper, b_paper, att_src_ap, att_dst_ap, att_src_pa, att_dst_pa, q, Wk, bk, W_out, b_out):
    raise NotImplementedError("write your pallas kernel here")


# speedup vs baseline: 50.1578x; 50.1578x over previous
import functools

import jax
import jax.numpy as jnp
from jax import lax
from jax.experimental import pallas as pl
from jax.experimental.pallas import tpu as pltpu
from jax.experimental.pallas import tpu_sc as plsc

N = 10000
E = 320000
H = 8
DH = 16
DHID = 128
DOUT = 64

NC = 2
NS = 16
NW = NC * NS
EPW = E // NW
CHUNK = 80
NCHUNK = EPW // CHUNK
STRIPE = 624
RBLK = 2000


def _tc_prep_body(xp_ref, xa_ref, wp_ref, bp_ref, wa_ref, ba_ref,
                  p2s_ref, p2d_ref, hp_ref, ads_ref, add_ref, crow_ref):
    hp = jnp.dot(xp_ref[...], wp_ref[...],
                 preferred_element_type=jnp.float32) + bp_ref[...]
    hp_ref[...] = hp
    ads = jnp.dot(hp, p2s_ref[...], preferred_element_type=jnp.float32)
    ads_ref[...] = ads
    ha = jnp.dot(xa_ref[...], wa_ref[...],
                 preferred_element_type=jnp.float32) + ba_ref[...]
    add = jnp.dot(ha, p2d_ref[...], preferred_element_type=jnp.float32)
    add_ref[...] = add
    ms = jnp.max(ads, axis=0, keepdims=True)
    md = jnp.max(add, axis=0, keepdims=True)
    rid = lax.broadcasted_iota(jnp.int32, (8, 16), 0)
    cur = jnp.where(rid == 0, ms, jnp.where(rid == 1, md, -jnp.inf))

    @pl.when(pl.program_id(0) == 0)
    def _():
        crow_ref[...] = cur

    @pl.when(pl.program_id(0) != 0)
    def _():
        crow_ref[...] = jnp.maximum(crow_ref[...], cur)


def _sc_body(src_hbm, dst_hbm, ads_hbm, add_hbm, crow_hbm, hp_hbm,
             acc_out, s_out,
             srcv, dstv, asbuf, adbuf, evbuf, hpbuf, crowv, acc_sh, s_sh):
    cid = lax.axis_index("c")
    sid = lax.axis_index("s")
    wid = sid * NC + cid

    def zrow(e, carry):
        for k in range(H):
            hpbuf[e, pl.ds(k * 16, 16)] = jnp.zeros((16,), jnp.float32)
        evbuf[e, :] = jnp.zeros((16,), jnp.float32)
        return carry

    lax.fori_loop(0, CHUNK, zrow, 0)
    base = sid * STRIPE
    off = 0
    for sz in (80, 80, 80, 80, 80, 80, 80, 64):
        pltpu.sync_copy(hpbuf.at[pl.ds(0, sz)], acc_sh.at[pl.ds(base + off, sz)])
        pltpu.sync_copy(evbuf.at[pl.ds(0, sz)], s_sh.at[pl.ds(base + off, sz)])
        off += sz

    @pl.when(sid == NS - 1)
    def _():
        pltpu.sync_copy(hpbuf.at[pl.ds(0, 16)], acc_sh.at[pl.ds(NS * STRIPE, 16)])
        pltpu.sync_copy(evbuf.at[pl.ds(0, 16)], s_sh.at[pl.ds(NS * STRIPE, 16)])

    plsc.subcore_barrier()

    pltpu.sync_copy(crow_hbm, crowv)
    c = crowv[0, :] + crowv[1, :]
    c16 = jnp.where(c >= 0.0, c, 0.2 * c)

    def chunk_body(i, carry):
        ebase = wid * EPW + i * CHUNK
        pltpu.sync_copy(src_hbm.at[pl.ds(ebase, CHUNK)], srcv)
        pltpu.sync_copy(dst_hbm.at[pl.ds(ebase, CHUNK)], dstv)
        pltpu.sync_copy(ads_hbm.at[srcv], asbuf)
        pltpu.sync_copy(add_hbm.at[dstv], adbuf)
        pltpu.sync_copy(hp_hbm.at[srcv], hpbuf)

        def ev_body(e, cc):
            a = asbuf[e, :] + adbuf[e, :]
            a = jnp.where(a >= 0.0, a, 0.2 * a)
            ev = jnp.exp(a - c16)
            evbuf[e, :] = ev
            for k in range(H):
                m = ev.at[jnp.full((16,), k, jnp.int32)].get(
                    mode="promise_in_bounds")
                hpbuf[e, pl.ds(k * 16, 16)] = hpbuf[e, pl.ds(k * 16, 16)] * m
            return cc

        lax.fori_loop(0, CHUNK, ev_body, 0)
        pltpu.sync_copy(evbuf, s_sh.at[dstv], add=True)
        pltpu.sync_copy(hpbuf, acc_sh.at[dstv], add=True)
        return carry

    lax.fori_loop(0, NCHUNK, chunk_body, 0)
    plsc.subcore_barrier()

    pltpu.sync_copy(acc_sh.at[pl.ds(base, STRIPE)],
                    acc_out.at[cid, pl.ds(base, STRIPE)])
    pltpu.sync_copy(s_sh.at[pl.ds(base, STRIPE)],
                    s_out.at[cid, pl.ds(base, STRIPE)])

    @pl.when(sid == NS - 1)
    def _():
        pltpu.sync_copy(acc_sh.at[pl.ds(NS * STRIPE, 16)],
                        acc_out.at[cid, pl.ds(NS * STRIPE, 16)])
        pltpu.sync_copy(s_sh.at[pl.ds(NS * STRIPE, 16)],
                        s_out.at[cid, pl.ds(NS * STRIPE, 16)])


def _tc_final_body(accp_ref, sp_ref, r2_ref, wout_ref, bout_ref, out_ref):
    acc = accp_ref[0] + accp_ref[1]
    s = sp_ref[0] + sp_ref[1]
    srep = jnp.dot(s, r2_ref[...], preferred_element_type=jnp.float32)
    y = jnp.maximum(acc, 0.0) / (srep + 1e-16)
    out_ref[...] = jnp.dot(y, wout_ref[...],
                           preferred_element_type=jnp.float32) + bout_ref[...]


def kernel(x_author, x_paper, edge_index_ap, edge_index_pa,
           W_author, b_author, W_paper, b_paper,
           att_src_ap, att_dst_ap, att_src_pa, att_dst_pa,
           q, Wk, bk, W_out, b_out):
    f32 = jnp.float32
    src = edge_index_pa[0]
    dst = edge_index_pa[1]

    def place(att):
        flat = att.reshape(DHID)
        m = jnp.zeros((DHID, H), f32).at[
            jnp.arange(DHID), jnp.arange(DHID) // DH].set(flat)
        return jnp.concatenate([m, m], axis=1)

    p2s = place(att_src_pa)
    p2d = place(att_dst_pa)
    cols = jnp.arange(DHID) // DH
    rows = jnp.arange(16)
    r2 = ((rows[:, None] == cols[None, :]) & (rows[:, None] < H)).astype(f32)

    grid = N // RBLK
    hp, ads, add, crow = pl.pallas_call(
        _tc_prep_body,
        grid=(grid,),
        in_specs=[
            pl.BlockSpec((RBLK, DHID), lambda i: (i, 0)),
            pl.BlockSpec((RBLK, DHID), lambda i: (i, 0)),
            pl.BlockSpec((DHID, DHID), lambda i: (0, 0)),
            pl.BlockSpec((1, DHID), lambda i: (0, 0)),
            pl.BlockSpec((DHID, DHID), lambda i: (0, 0)),
            pl.BlockSpec((1, DHID), lambda i: (0, 0)),
            pl.BlockSpec((DHID, 16), lambda i: (0, 0)),
            pl.BlockSpec((DHID, 16), lambda i: (0, 0)),
        ],
        out_specs=[
            pl.BlockSpec((RBLK, DHID), lambda i: (i, 0)),
            pl.BlockSpec((RBLK, 16), lambda i: (i, 0)),
            pl.BlockSpec((RBLK, 16), lambda i: (i, 0)),
            pl.BlockSpec((8, 16), lambda i: (0, 0)),
        ],
        out_shape=[
            jax.ShapeDtypeStruct((N, DHID), f32),
            jax.ShapeDtypeStruct((N, 16), f32),
            jax.ShapeDtypeStruct((N, 16), f32),
            jax.ShapeDtypeStruct((8, 16), f32),
        ],
        compiler_params=pltpu.CompilerParams(
            dimension_semantics=("arbitrary",)),
    )(x_paper, x_author, W_paper, b_paper.reshape(1, DHID),
      W_author, b_author.reshape(1, DHID), p2s, p2d)

    sc_edge = pl.kernel(
        _sc_body,
        out_type=[
            jax.ShapeDtypeStruct((NC, N, DHID), f32),
            jax.ShapeDtypeStruct((NC, N, 16), f32),
        ],
        mesh=plsc.VectorSubcoreMesh(core_axis_name="c", subcore_axis_name="s"),
        compiler_params=pltpu.CompilerParams(use_tc_tiling_on_sc=False),
        scratch_types=[
            pltpu.VMEM((CHUNK,), jnp.int32),
            pltpu.VMEM((CHUNK,), jnp.int32),
            pltpu.VMEM((CHUNK, 16), f32),
            pltpu.VMEM((CHUNK, 16), f32),
            pltpu.VMEM((CHUNK, 16), f32),
            pltpu.VMEM((CHUNK, DHID), f32),
            pltpu.VMEM((8, 16), f32),
            pltpu.VMEM_SHARED((N, DHID), f32),
            pltpu.VMEM_SHARED((N, 16), f32),
        ],
    )
    accp, sp = sc_edge(src, dst, ads, add, crow, hp)

    out = pl.pallas_call(
        _tc_final_body,
        grid=(grid,),
        in_specs=[
            pl.BlockSpec((NC, RBLK, DHID), lambda i: (0, i, 0)),
            pl.BlockSpec((NC, RBLK, 16), lambda i: (0, i, 0)),
            pl.BlockSpec((16, DHID), lambda i: (0, 0)),
            pl.BlockSpec((DHID, DOUT), lambda i: (0, 0)),
            pl.BlockSpec((1, DOUT), lambda i: (0, 0)),
        ],
        out_specs=pl.BlockSpec((RBLK, DOUT), lambda i: (i, 0)),
        out_shape=jax.ShapeDtypeStruct((N, DOUT), f32),
        compiler_params=pltpu.CompilerParams(
            dimension_semantics=("arbitrary",)),
    )(accp, sp, r2, W_out, b_out.reshape(1, DOUT))
    return out


# speedup vs baseline: 81.1710x; 1.6183x over previous
import functools

import jax
import jax.numpy as jnp
from jax import lax
from jax.experimental import pallas as pl
from jax.experimental.pallas import tpu as pltpu
from jax.experimental.pallas import tpu_sc as plsc

N = 10000
E = 320000
H = 8
DH = 16
DHID = 128
DOUT = 64

NC = 2
NS = 16
NW = NC * NS
EPW = E // NW
CHUNK = 80
NCHUNK = EPW // CHUNK
STRIPE = 624
RBLK = 2000


def _tc_prep_body(xp_ref, xa_ref, wp_ref, bp_ref, wa_ref, ba_ref,
                  p2s_ref, p2d_ref, hp_ref, ads_ref, add_ref, crow_ref):
    hp = jnp.dot(xp_ref[...], wp_ref[...],
                 preferred_element_type=jnp.float32) + bp_ref[...]
    hp_ref[...] = hp
    ads = jnp.dot(hp, p2s_ref[...], preferred_element_type=jnp.float32)
    ads_ref[...] = ads
    ha = jnp.dot(xa_ref[...], wa_ref[...],
                 preferred_element_type=jnp.float32) + ba_ref[...]
    add = jnp.dot(ha, p2d_ref[...], preferred_element_type=jnp.float32)
    add_ref[...] = add
    ms = jnp.max(ads, axis=0, keepdims=True)
    md = jnp.max(add, axis=0, keepdims=True)
    rid = lax.broadcasted_iota(jnp.int32, (8, 16), 0)
    cur = jnp.where(rid == 0, ms, jnp.where(rid == 1, md, -jnp.inf))

    @pl.when(pl.program_id(0) == 0)
    def _():
        crow_ref[...] = cur

    @pl.when(pl.program_id(0) != 0)
    def _():
        crow_ref[...] = jnp.maximum(crow_ref[...], cur)


def _sc_body(src_hbm, dst_hbm, ads_hbm, add_hbm, crow_hbm, hp_hbm,
             acc_out, s_out, *scr):
    srcv = scr[0:2]
    dstv = scr[2:4]
    dstsc = scr[4:6]
    asbuf = scr[6:8]
    adbuf = scr[8:10]
    evbuf = scr[10:12]
    hpbuf = scr[12:14]
    isem = scr[14:16]
    gsem = scr[16:18]
    ssem = scr[18:20]
    crowv, acc_sh, s_sh = scr[20], scr[21], scr[22]

    cid = lax.axis_index("c")
    sid = lax.axis_index("s")
    wid = sid * NC + cid

    def zrow(e, carry):
        for k in range(H):
            hpbuf[0][e, pl.ds(k * 16, 16)] = jnp.zeros((16,), jnp.float32)
        evbuf[0][e, :] = jnp.zeros((16,), jnp.float32)
        return carry

    lax.fori_loop(0, CHUNK, zrow, 0)
    base = sid * STRIPE
    off = 0
    for sz in (80, 80, 80, 80, 80, 80, 80, 64):
        pltpu.sync_copy(hpbuf[0].at[pl.ds(0, sz)], acc_sh.at[pl.ds(base + off, sz)])
        pltpu.sync_copy(evbuf[0].at[pl.ds(0, sz)], s_sh.at[pl.ds(base + off, sz)])
        off += sz

    @pl.when(sid == NS - 1)
    def _():
        pltpu.sync_copy(hpbuf[0].at[pl.ds(0, 16)], acc_sh.at[pl.ds(NS * STRIPE, 16)])
        pltpu.sync_copy(evbuf[0].at[pl.ds(0, 16)], s_sh.at[pl.ds(NS * STRIPE, 16)])

    plsc.subcore_barrier()

    pltpu.sync_copy(crow_hbm, crowv)
    c = crowv[0, :] + crowv[1, :]
    c16 = jnp.where(c >= 0.0, c, 0.2 * c)

    def fire_idx(c, j):
        @pl.when(c < NCHUNK)
        def _():
            ebase = wid * EPW + c * CHUNK
            pltpu.async_copy(src_hbm.at[pl.ds(ebase, CHUNK)], srcv[j], isem[j])
            pltpu.async_copy(dst_hbm.at[pl.ds(ebase, CHUNK)], dstv[j], isem[j])

    def wait_idx(j):
        pltpu.make_async_copy(src_hbm.at[pl.ds(0, CHUNK)], srcv[j], isem[j]).wait()
        pltpu.make_async_copy(dst_hbm.at[pl.ds(0, CHUNK)], dstv[j], isem[j]).wait()

    def fire_gathers(j):
        pltpu.async_copy(ads_hbm.at[srcv[j]], asbuf[j], gsem[j])
        pltpu.async_copy(add_hbm.at[dstv[j]], adbuf[j], gsem[j])
        pltpu.async_copy(hp_hbm.at[srcv[j]], hpbuf[j], gsem[j])

    def wait_gathers(j):
        pltpu.make_async_copy(ads_hbm.at[srcv[j]], asbuf[j], gsem[j]).wait()
        pltpu.make_async_copy(add_hbm.at[dstv[j]], adbuf[j], gsem[j]).wait()
        pltpu.make_async_copy(hp_hbm.at[srcv[j]], hpbuf[j], gsem[j]).wait()

    def fire_scatters(j):
        pltpu.async_copy(evbuf[j], s_sh.at[dstsc[j]], ssem[j], add=True)
        pltpu.async_copy(hpbuf[j], acc_sh.at[dstsc[j]], ssem[j], add=True)

    def wait_scatters(j):
        pltpu.make_async_copy(evbuf[j], s_sh.at[dstsc[j]], ssem[j]).wait()
        pltpu.make_async_copy(hpbuf[j], acc_sh.at[dstsc[j]], ssem[j]).wait()

    def copy_dst(j):
        for t in range(CHUNK // 16):
            dstsc[j][pl.ds(t * 16, 16)] = dstv[j][pl.ds(t * 16, 16)]

    def compute(j):
        asb, adb, evb, hpb = asbuf[j], adbuf[j], evbuf[j], hpbuf[j]

        def ev_body(e, cc):
            a = asb[e, :] + adb[e, :]
            a = jnp.where(a >= 0.0, a, 0.2 * a)
            ev = jnp.exp(a - c16)
            evb[e, :] = ev
            for k in range(H):
                m = ev.at[jnp.full((16,), k, jnp.int32)].get(
                    mode="promise_in_bounds")
                hpb[e, pl.ds(k * 16, 16)] = hpb[e, pl.ds(k * 16, 16)] * m
            return cc

        lax.fori_loop(0, CHUNK, ev_body, 0)

    for j in range(2):
        fire_idx(jnp.int32(j), j)
    wait_idx(0)
    fire_gathers(0)

    def pipe_body(p, carry):
        c0 = p * 2
        for j in range(2):
            cj = c0 + j
            jn = (j + 1) % 2
            wait_gathers(j)
            copy_dst(j)
            fire_idx(cj + 2, j)
            compute(j)
            fire_scatters(j)

            @pl.when(cj + 1 >= 2)
            def _():
                wait_scatters(jn)

            wait_idx(jn)
            fire_gathers(jn)
        return carry

    lax.fori_loop(0, (NCHUNK - 1) // 2, pipe_body, 0)

    wait_gathers(0)
    copy_dst(0)
    compute(0)
    fire_scatters(0)
    for j in (1, 0):
        wait_scatters(j)
    plsc.subcore_barrier()

    pltpu.sync_copy(acc_sh.at[pl.ds(base, STRIPE)],
                    acc_out.at[cid, pl.ds(base, STRIPE)])
    pltpu.sync_copy(s_sh.at[pl.ds(base, STRIPE)],
                    s_out.at[cid, pl.ds(base, STRIPE)])

    @pl.when(sid == NS - 1)
    def _():
        pltpu.sync_copy(acc_sh.at[pl.ds(NS * STRIPE, 16)],
                        acc_out.at[cid, pl.ds(NS * STRIPE, 16)])
        pltpu.sync_copy(s_sh.at[pl.ds(NS * STRIPE, 16)],
                        s_out.at[cid, pl.ds(NS * STRIPE, 16)])


def _tc_final_body(accp_ref, sp_ref, r2_ref, wout_ref, bout_ref, out_ref):
    acc = accp_ref[0] + accp_ref[1]
    s = sp_ref[0] + sp_ref[1]
    srep = jnp.dot(s, r2_ref[...], preferred_element_type=jnp.float32)
    y = jnp.maximum(acc, 0.0) / (srep + 1e-16)
    out_ref[...] = jnp.dot(y, wout_ref[...],
                           preferred_element_type=jnp.float32) + bout_ref[...]


def kernel(x_author, x_paper, edge_index_ap, edge_index_pa,
           W_author, b_author, W_paper, b_paper,
           att_src_ap, att_dst_ap, att_src_pa, att_dst_pa,
           q, Wk, bk, W_out, b_out):
    f32 = jnp.float32
    src = edge_index_pa[0]
    dst = edge_index_pa[1]

    def place(att):
        flat = att.reshape(DHID)
        m = jnp.zeros((DHID, H), f32).at[
            jnp.arange(DHID), jnp.arange(DHID) // DH].set(flat)
        return jnp.concatenate([m, m], axis=1)

    p2s = place(att_src_pa)
    p2d = place(att_dst_pa)
    cols = jnp.arange(DHID) // DH
    rows = jnp.arange(16)
    r2 = ((rows[:, None] == cols[None, :]) & (rows[:, None] < H)).astype(f32)

    grid = N // RBLK
    hp, ads, add, crow = pl.pallas_call(
        _tc_prep_body,
        grid=(grid,),
        in_specs=[
            pl.BlockSpec((RBLK, DHID), lambda i: (i, 0)),
            pl.BlockSpec((RBLK, DHID), lambda i: (i, 0)),
            pl.BlockSpec((DHID, DHID), lambda i: (0, 0)),
            pl.BlockSpec((1, DHID), lambda i: (0, 0)),
            pl.BlockSpec((DHID, DHID), lambda i: (0, 0)),
            pl.BlockSpec((1, DHID), lambda i: (0, 0)),
            pl.BlockSpec((DHID, 16), lambda i: (0, 0)),
            pl.BlockSpec((DHID, 16), lambda i: (0, 0)),
        ],
        out_specs=[
            pl.BlockSpec((RBLK, DHID), lambda i: (i, 0)),
            pl.BlockSpec((RBLK, 16), lambda i: (i, 0)),
            pl.BlockSpec((RBLK, 16), lambda i: (i, 0)),
            pl.BlockSpec((8, 16), lambda i: (0, 0)),
        ],
        out_shape=[
            jax.ShapeDtypeStruct((N, DHID), f32),
            jax.ShapeDtypeStruct((N, 16), f32),
            jax.ShapeDtypeStruct((N, 16), f32),
            jax.ShapeDtypeStruct((8, 16), f32),
        ],
        compiler_params=pltpu.CompilerParams(
            dimension_semantics=("arbitrary",)),
    )(x_paper, x_author, W_paper, b_paper.reshape(1, DHID),
      W_author, b_author.reshape(1, DHID), p2s, p2d)

    sc_edge = pl.kernel(
        _sc_body,
        out_type=[
            jax.ShapeDtypeStruct((NC, N, DHID), f32),
            jax.ShapeDtypeStruct((NC, N, 16), f32),
        ],
        mesh=plsc.VectorSubcoreMesh(core_axis_name="c", subcore_axis_name="s"),
        compiler_params=pltpu.CompilerParams(use_tc_tiling_on_sc=False),
        scratch_types=(
            [pltpu.VMEM((CHUNK,), jnp.int32)] * 6
            + [pltpu.VMEM((CHUNK, 16), f32)] * 6
            + [pltpu.VMEM((CHUNK, DHID), f32)] * 2
            + [pltpu.SemaphoreType.DMA] * 6
            + [
                pltpu.VMEM((8, 16), f32),
                pltpu.VMEM_SHARED((N, DHID), f32),
                pltpu.VMEM_SHARED((N, 16), f32),
            ]
        ),
    )
    accp, sp = sc_edge(src, dst, ads, add, crow, hp)

    out = pl.pallas_call(
        _tc_final_body,
        grid=(grid,),
        in_specs=[
            pl.BlockSpec((NC, RBLK, DHID), lambda i: (0, i, 0)),
            pl.BlockSpec((NC, RBLK, 16), lambda i: (0, i, 0)),
            pl.BlockSpec((16, DHID), lambda i: (0, 0)),
            pl.BlockSpec((DHID, DOUT), lambda i: (0, 0)),
            pl.BlockSpec((1, DOUT), lambda i: (0, 0)),
        ],
        out_specs=pl.BlockSpec((RBLK, DOUT), lambda i: (i, 0)),
        out_shape=jax.ShapeDtypeStruct((N, DOUT), f32),
        compiler_params=pltpu.CompilerParams(
            dimension_semantics=("arbitrary",)),
    )(accp, sp, r2, W_out, b_out.reshape(1, DOUT))
    return out


# speedup vs baseline: 124.8152x; 1.5377x over previous
import functools

import jax
import jax.numpy as jnp
from jax import lax
from jax.experimental import pallas as pl
from jax.experimental.pallas import tpu as pltpu
from jax.experimental.pallas import tpu_sc as plsc

N = 10000
E = 320000
H = 8
DH = 16
DHID = 128
DOUT = 64

NC = 2
NS = 16
NW = NC * NS
EPW = E // NW
CHUNK = 80
NCHUNK = EPW // CHUNK
STRIPE = 624
RBLK = 2000


def _tc_prep_body(xp_ref, xa_ref, wp_ref, bp_ref, wa_ref, ba_ref,
                  p2s_ref, p2d_ref, hp_ref, ads_ref, add_ref, crow_ref):
    hp = jnp.dot(xp_ref[...], wp_ref[...],
                 preferred_element_type=jnp.float32) + bp_ref[...]
    hp_ref[...] = hp
    ads = jnp.dot(hp, p2s_ref[...], preferred_element_type=jnp.float32)
    ads_ref[...] = ads
    ha = jnp.dot(xa_ref[...], wa_ref[...],
                 preferred_element_type=jnp.float32) + ba_ref[...]
    add = jnp.dot(ha, p2d_ref[...], preferred_element_type=jnp.float32)
    add_ref[...] = add
    ms = jnp.max(ads, axis=0, keepdims=True)
    md = jnp.max(add, axis=0, keepdims=True)
    rid = lax.broadcasted_iota(jnp.int32, (8, 16), 0)
    cur = jnp.where(rid == 0, ms, jnp.where(rid == 1, md, -jnp.inf))

    @pl.when(pl.program_id(0) == 0)
    def _():
        crow_ref[...] = cur

    @pl.when(pl.program_id(0) != 0)
    def _():
        crow_ref[...] = jnp.maximum(crow_ref[...], cur)


def _sc_body(src_hbm, dst_hbm, ads_hbm, add_hbm, crow_hbm, hp_hbm,
             acc_out, s_out, *scr):
    srcv = scr[0:2]
    dstv = scr[2:4]
    dstsc = scr[4:6]
    asbuf = scr[6:8]
    adbuf = scr[8:10]
    evbuf = scr[10:12]
    hpbuf = scr[12:14]
    isem = scr[14:16]
    gsem = scr[16:18]
    ssem = scr[18:20]
    crowv, acc_sh, s_sh = scr[20], scr[21], scr[22]

    cid = lax.axis_index("c")
    sid = lax.axis_index("s")
    wid = sid * NC + cid

    def zrow(e, carry):
        for k in range(H):
            hpbuf[0][e, pl.ds(k * 16, 16)] = jnp.zeros((16,), jnp.float32)
        evbuf[0][e, :] = jnp.zeros((16,), jnp.float32)
        return carry

    lax.fori_loop(0, CHUNK, zrow, 0)
    base = sid * STRIPE
    off = 0
    for sz in (80, 80, 80, 80, 80, 80, 80, 64):
        pltpu.sync_copy(hpbuf[0].at[pl.ds(0, sz)], acc_sh.at[pl.ds(base + off, sz)])
        pltpu.sync_copy(evbuf[0].at[pl.ds(0, sz)], s_sh.at[pl.ds(base + off, sz)])
        off += sz

    @pl.when(sid == NS - 1)
    def _():
        pltpu.sync_copy(hpbuf[0].at[pl.ds(0, 16)], acc_sh.at[pl.ds(NS * STRIPE, 16)])
        pltpu.sync_copy(evbuf[0].at[pl.ds(0, 16)], s_sh.at[pl.ds(NS * STRIPE, 16)])

    plsc.subcore_barrier()

    pltpu.sync_copy(crow_hbm, crowv)
    c = crowv[0, :] + crowv[1, :]
    c16 = jnp.where(c >= 0.0, c, 0.2 * c)

    def fire_idx(c, j):
        @pl.when(c < NCHUNK)
        def _():
            ebase = wid * EPW + c * CHUNK
            pltpu.async_copy(src_hbm.at[pl.ds(ebase, CHUNK)], srcv[j], isem[j])
            pltpu.async_copy(dst_hbm.at[pl.ds(ebase, CHUNK)], dstv[j], isem[j])

    def wait_idx(j):
        pltpu.make_async_copy(src_hbm.at[pl.ds(0, CHUNK)], srcv[j], isem[j]).wait()
        pltpu.make_async_copy(dst_hbm.at[pl.ds(0, CHUNK)], dstv[j], isem[j]).wait()

    def fire_gathers(j):
        pltpu.async_copy(ads_hbm.at[srcv[j]], asbuf[j], gsem[j])
        pltpu.async_copy(add_hbm.at[dstv[j]], adbuf[j], gsem[j])
        pltpu.async_copy(hp_hbm.at[srcv[j]], hpbuf[j], gsem[j])

    def wait_gathers(j):
        pltpu.make_async_copy(ads_hbm.at[srcv[j]], asbuf[j], gsem[j]).wait()
        pltpu.make_async_copy(add_hbm.at[dstv[j]], adbuf[j], gsem[j]).wait()
        pltpu.make_async_copy(hp_hbm.at[srcv[j]], hpbuf[j], gsem[j]).wait()

    def fire_scatters(j):
        pltpu.async_copy(evbuf[j], s_sh.at[dstsc[j]], ssem[j], add=True)
        pltpu.async_copy(hpbuf[j], acc_sh.at[dstsc[j]], ssem[j], add=True)

    def wait_scatters(j):
        pltpu.make_async_copy(evbuf[j], s_sh.at[dstsc[j]], ssem[j]).wait()
        pltpu.make_async_copy(hpbuf[j], acc_sh.at[dstsc[j]], ssem[j]).wait()

    def copy_dst(j):
        for t in range(CHUNK // 16):
            dstsc[j][pl.ds(t * 16, 16)] = dstv[j][pl.ds(t * 16, 16)]

    def compute(j):
        asb, adb, evb, hpb = asbuf[j], adbuf[j], evbuf[j], hpbuf[j]

        @plsc.parallel_loop(0, CHUNK, step=1, unroll=4)
        def _(e):
            a = asb[e, :] + adb[e, :]
            a = jnp.where(a >= 0.0, a, 0.2 * a)
            ev = jnp.exp(a - c16)
            evb[e, :] = ev
            for k in range(H):
                m = ev.at[jnp.full((16,), k, jnp.int32)].get(
                    mode="promise_in_bounds")
                hpb[e, pl.ds(k * 16, 16)] = hpb[e, pl.ds(k * 16, 16)] * m

    for j in range(2):
        fire_idx(jnp.int32(j), j)
    wait_idx(0)
    fire_gathers(0)

    def pipe_body(p, carry):
        c0 = p * 2
        for j in range(2):
            cj = c0 + j
            jn = (j + 1) % 2
            wait_gathers(j)
            copy_dst(j)
            fire_idx(cj + 2, j)
            compute(j)
            fire_scatters(j)

            @pl.when(cj + 1 >= 2)
            def _():
                wait_scatters(jn)

            wait_idx(jn)
            fire_gathers(jn)
        return carry

    lax.fori_loop(0, (NCHUNK - 1) // 2, pipe_body, 0)

    wait_gathers(0)
    copy_dst(0)
    compute(0)
    fire_scatters(0)
    for j in (1, 0):
        wait_scatters(j)
    plsc.subcore_barrier()

    pltpu.sync_copy(acc_sh.at[pl.ds(base, STRIPE)],
                    acc_out.at[cid, pl.ds(base, STRIPE)])
    pltpu.sync_copy(s_sh.at[pl.ds(base, STRIPE)],
                    s_out.at[cid, pl.ds(base, STRIPE)])

    @pl.when(sid == NS - 1)
    def _():
        pltpu.sync_copy(acc_sh.at[pl.ds(NS * STRIPE, 16)],
                        acc_out.at[cid, pl.ds(NS * STRIPE, 16)])
        pltpu.sync_copy(s_sh.at[pl.ds(NS * STRIPE, 16)],
                        s_out.at[cid, pl.ds(NS * STRIPE, 16)])


def _tc_final_body(accp_ref, sp_ref, r2_ref, wout_ref, bout_ref, out_ref):
    acc = accp_ref[0] + accp_ref[1]
    s = sp_ref[0] + sp_ref[1]
    srep = jnp.dot(s, r2_ref[...], preferred_element_type=jnp.float32)
    y = jnp.maximum(acc, 0.0) / (srep + 1e-16)
    out_ref[...] = jnp.dot(y, wout_ref[...],
                           preferred_element_type=jnp.float32) + bout_ref[...]


def kernel(x_author, x_paper, edge_index_ap, edge_index_pa,
           W_author, b_author, W_paper, b_paper,
           att_src_ap, att_dst_ap, att_src_pa, att_dst_pa,
           q, Wk, bk, W_out, b_out):
    f32 = jnp.float32
    src = edge_index_pa[0]
    dst = edge_index_pa[1]

    def place(att):
        flat = att.reshape(DHID)
        m = jnp.zeros((DHID, H), f32).at[
            jnp.arange(DHID), jnp.arange(DHID) // DH].set(flat)
        return jnp.concatenate([m, m], axis=1)

    p2s = place(att_src_pa)
    p2d = place(att_dst_pa)
    cols = jnp.arange(DHID) // DH
    rows = jnp.arange(16)
    r2 = ((rows[:, None] == cols[None, :]) & (rows[:, None] < H)).astype(f32)

    grid = N // RBLK
    hp, ads, add, crow = pl.pallas_call(
        _tc_prep_body,
        grid=(grid,),
        in_specs=[
            pl.BlockSpec((RBLK, DHID), lambda i: (i, 0)),
            pl.BlockSpec((RBLK, DHID), lambda i: (i, 0)),
            pl.BlockSpec((DHID, DHID), lambda i: (0, 0)),
            pl.BlockSpec((1, DHID), lambda i: (0, 0)),
            pl.BlockSpec((DHID, DHID), lambda i: (0, 0)),
            pl.BlockSpec((1, DHID), lambda i: (0, 0)),
            pl.BlockSpec((DHID, 16), lambda i: (0, 0)),
            pl.BlockSpec((DHID, 16), lambda i: (0, 0)),
        ],
        out_specs=[
            pl.BlockSpec((RBLK, DHID), lambda i: (i, 0)),
            pl.BlockSpec((RBLK, 16), lambda i: (i, 0)),
            pl.BlockSpec((RBLK, 16), lambda i: (i, 0)),
            pl.BlockSpec((8, 16), lambda i: (0, 0)),
        ],
        out_shape=[
            jax.ShapeDtypeStruct((N, DHID), f32),
            jax.ShapeDtypeStruct((N, 16), f32),
            jax.ShapeDtypeStruct((N, 16), f32),
            jax.ShapeDtypeStruct((8, 16), f32),
        ],
        compiler_params=pltpu.CompilerParams(
            dimension_semantics=("arbitrary",)),
    )(x_paper, x_author, W_paper, b_paper.reshape(1, DHID),
      W_author, b_author.reshape(1, DHID), p2s, p2d)

    sc_edge = pl.kernel(
        _sc_body,
        out_type=[
            jax.ShapeDtypeStruct((NC, N, DHID), f32),
            jax.ShapeDtypeStruct((NC, N, 16), f32),
        ],
        mesh=plsc.VectorSubcoreMesh(core_axis_name="c", subcore_axis_name="s"),
        compiler_params=pltpu.CompilerParams(use_tc_tiling_on_sc=False),
        scratch_types=(
            [pltpu.VMEM((CHUNK,), jnp.int32)] * 6
            + [pltpu.VMEM((CHUNK, 16), f32)] * 6
            + [pltpu.VMEM((CHUNK, DHID), f32)] * 2
            + [pltpu.SemaphoreType.DMA] * 6
            + [
                pltpu.VMEM((8, 16), f32),
                pltpu.VMEM_SHARED((N, DHID), f32),
                pltpu.VMEM_SHARED((N, 16), f32),
            ]
        ),
    )
    accp, sp = sc_edge(src, dst, ads, add, crow, hp)

    out = pl.pallas_call(
        _tc_final_body,
        grid=(grid,),
        in_specs=[
            pl.BlockSpec((NC, RBLK, DHID), lambda i: (0, i, 0)),
            pl.BlockSpec((NC, RBLK, 16), lambda i: (0, i, 0)),
            pl.BlockSpec((16, DHID), lambda i: (0, 0)),
            pl.BlockSpec((DHID, DOUT), lambda i: (0, 0)),
            pl.BlockSpec((1, DOUT), lambda i: (0, 0)),
        ],
        out_specs=pl.BlockSpec((RBLK, DOUT), lambda i: (i, 0)),
        out_shape=jax.ShapeDtypeStruct((N, DOUT), f32),
        compiler_params=pltpu.CompilerParams(
            dimension_semantics=("arbitrary",)),
    )(accp, sp, r2, W_out, b_out.reshape(1, DOUT))
    return out
